# baseline (speedup 1.0000x reference)
import jax
import jax.numpy as jnp
from jax.experimental import pallas as pl

B, V, C = 64, 32, 64
R = B * V
CHUNK = 256
NCH = R // CHUNK
_NEG = -jnp.inf


def _shl(x, k, pad):
    return jnp.concatenate(
        [x[:, k:], jnp.full((x.shape[0], k), pad, x.dtype)], axis=1)


def _shr(x, k, pad):
    return jnp.concatenate(
        [jnp.full((x.shape[0], k), pad, x.dtype), x[:, : x.shape[1] - k]],
        axis=1)


def _vae_kernel(x_ref, u_ref, iw_ref, s_ref):
    x = x_ref[:]
    u = u_ref[:]
    lane = jax.lax.broadcasted_iota(jnp.int32, (CHUNK, C), 1)
    valid = lane < C - 1

    m = _shl(x, 1, _NEG)
    for k in (1, 2, 4, 8, 16, 32):
        m = jnp.maximum(m, _shl(m, k, _NEG))

    ii = jax.lax.broadcasted_iota(jnp.int32, (CHUNK, C, C), 1)
    jj = jax.lax.broadcasted_iota(jnp.int32, (CHUNK, C, C), 2)
    d3 = jnp.where(jj > ii, x[:, None, :] - m[:, :, None], _NEG)
    ssum = jnp.sum(jnp.exp(d3), axis=-1)
    denom = jnp.log(ssum) + m
    sb = jnp.where(valid, x - denom, 0.0)

    sg = jax.nn.sigmoid(sb)
    sg_abs = jax.nn.sigmoid(jnp.abs(sb))
    cond = sb >= -1e-5
    safe_den = jnp.where(cond, 1.0, 1.0 - 2.0 * sg)
    bzm = jnp.where(cond, 0.0, (1.0 - sg) ** 2 / safe_den)
    for k in (1, 2, 4, 8, 16, 32):
        bzm = bzm * _shr(bzm, k, 1.0)
    iw_ref[:] = _shr(bzm, 1, 1.0) * jnp.where(valid, sg_abs, 1.0)

    hit = (u < sg) & valid
    s_ref[:] = jnp.min(jnp.where(hit, lane, C - 1), axis=1, keepdims=True)


@jax.jit
def kernel(encoder_logits, u_noise):
    x = encoder_logits.reshape(R, C)
    u = u_noise.reshape(R, C)
    iw, samp = pl.pallas_call(
        _vae_kernel,
        grid=(NCH,),
        in_specs=[
            pl.BlockSpec((CHUNK, C), lambda i: (i, 0)),
            pl.BlockSpec((CHUNK, C), lambda i: (i, 0)),
        ],
        out_specs=(
            pl.BlockSpec((CHUNK, C), lambda i: (i, 0)),
            pl.BlockSpec((CHUNK, 1), lambda i: (i, 0)),
        ),
        out_shape=(
            jax.ShapeDtypeStruct((R, C), jnp.float32),
            jax.ShapeDtypeStruct((R, 1), jnp.int32),
        ),
    )(x, u)
    return iw.reshape(B, V, C), samp.reshape(B, V)


# speedup vs baseline: 1.8954x; 1.8954x over previous
import jax
import jax.numpy as jnp
from jax.experimental import pallas as pl
from jax.experimental.pallas import tpu as pltpu

B, V, C = 64, 32, 64
R = B * V
RP = R // 2
W = 2 * C
KCAP = 128
_SHIFT = 16.0
_NEG = -jnp.inf


def _seg_shl1(x, pad):
    lane = jax.lax.broadcasted_iota(jnp.int32, x.shape, 1)
    y = jnp.concatenate(
        [x[:, 1:], jnp.full((x.shape[0], 1), pad, x.dtype)], axis=1)
    return jnp.where((lane & (C - 1)) == C - 1, pad, y)


def _seg_shr1(x, pad):
    lane = jax.lax.broadcasted_iota(jnp.int32, x.shape, 1)
    y = jnp.concatenate(
        [jnp.full((x.shape[0], 1), pad, x.dtype), x[:, :-1]], axis=1)
    return jnp.where((lane & (C - 1)) == 0, pad, y)


def _shl(x, k, pad):
    return jnp.concatenate(
        [x[:, k:], jnp.full((x.shape[0], k), pad, x.dtype)], axis=1)


def _shr(x, k, pad):
    return jnp.concatenate(
        [jnp.full((x.shape[0], k), pad, x.dtype), x[:, : x.shape[1] - k]],
        axis=1)


def _mm(a, b, precision):
    return jax.lax.dot_general(
        a, b, (((1,), (0,)), ((), ())),
        preferred_element_type=jnp.float32, precision=precision)


def _fast_kernel(x_ref, u_ref, iw_ref, s_ref, f_ref):
    x = x_ref[:]
    u = u_ref[:]
    lane = jax.lax.broadcasted_iota(jnp.int32, (RP, W), 1)
    valid = (lane & (C - 1)) < C - 1

    jj = jax.lax.broadcasted_iota(jnp.int32, (W, W), 0)
    ii = jax.lax.broadcasted_iota(jnp.int32, (W, W), 1)
    samehalf = (jj // C) == (ii // C)
    upper = ((jj > ii) & samehalf).astype(jnp.float32)
    lower_f = ((jj <= ii) & samehalf).astype(jnp.float32)
    lower_b = ((jj <= ii) & samehalf).astype(jnp.bfloat16)

    e = jnp.exp(x - _SHIFT)
    s = _mm(e, upper, jax.lax.Precision.HIGHEST)
    xs = _seg_shl1(x, 0.0)
    denom = xs + jnp.log(s * jnp.exp(_SHIFT - xs))
    sb = jnp.where(valid, x - denom, 0.0)

    sg = jax.nn.sigmoid(sb)
    sg_abs = jnp.maximum(sg, 1.0 - sg)
    cond = sb >= -1e-5
    ratio = (1.0 - sg) ** 2 / jnp.where(cond, 1.0, 1.0 - 2.0 * sg)
    lg = jnp.where(cond, -1e4, jnp.log(ratio))
    p = jnp.exp(_mm(lg, lower_f, jax.lax.Precision.HIGHEST))
    iw_ref[:] = _seg_shr1(p, 1.0) * jnp.where(valid, sg_abs, 1.0)

    h = (u < sg).astype(jnp.bfloat16)
    q = _mm(h, lower_b, jax.lax.Precision.DEFAULT)
    ind = (q == 0.0).astype(jnp.bfloat16)
    oj = jax.lax.broadcasted_iota(jnp.int32, (W, 2), 0)
    oc = jax.lax.broadcasted_iota(jnp.int32, (W, 2), 1)
    seg_of = ((oj // C) == oc)
    count = (seg_of & ((oj & (C - 1)) < C - 1)).astype(jnp.bfloat16)
    s_ref[:] = _mm(ind, count, jax.lax.Precision.DEFAULT).astype(jnp.int32)

    danger = valid & (((sb > -2e-2) & (sb < 9e-5))
                      | (jnp.abs(u - sg) < 1e-5))
    dcnt = _mm(danger.astype(jnp.bfloat16), seg_of.astype(jnp.bfloat16),
               jax.lax.Precision.DEFAULT)
    f_ref[:] = (dcnt > 0.0).astype(jnp.int32)


def _fix_kernel(idx_ref, x_ref, u_ref, iwf_ref, sampf_ref,
                iw_ref, samp_ref, xg_ref, ug_ref, iwg_ref, sg_ref):
    iw_ref[:] = iwf_ref[:]
    samp_ref[:] = sampf_ref[:]

    def gather(t, carry):
        r = idx_ref[t]
        xg_ref[pl.ds(t, 1), :] = x_ref[pl.ds(r, 1), :]
        ug_ref[pl.ds(t, 1), :] = u_ref[pl.ds(r, 1), :]
        return carry

    jax.lax.fori_loop(0, KCAP, gather, 0)

    x = xg_ref[:]
    u = ug_ref[:]
    lane = jax.lax.broadcasted_iota(jnp.int32, (KCAP, C), 1)
    valid = lane < C - 1

    tiled = jnp.broadcast_to(x[:, None, :], (KCAP, C, C))
    ii = jax.lax.broadcasted_iota(jnp.int32, (KCAP, C, C), 1)
    jj = jax.lax.broadcasted_iota(jnp.int32, (KCAP, C, C), 2)
    dl = jnp.where(jj <= ii, _NEG, tiled)
    amax = jnp.max(dl, axis=-1, keepdims=True)
    ssum = jnp.sum(jnp.exp(dl - amax), axis=-1)
    denom = jnp.log(ssum) + amax[:, :, 0]
    sb = jnp.where(valid, x - denom, 0.0)

    sg = jax.nn.sigmoid(sb)
    sg_abs = jax.nn.sigmoid(jnp.abs(sb))
    cond = sb >= -1e-5
    safe_den = jnp.where(cond, 1.0, 1.0 - 2.0 * sg)
    bzm = jnp.where(cond, 0.0, (1.0 - sg) ** 2 / safe_den)
    bzm = jnp.where(valid, bzm, 0.0)
    for k in (1, 2, 4, 8, 16, 32):
        bzm = bzm * _shr(bzm, k, 1.0)
    iwg_ref[:] = _shr(bzm, 1, 1.0) * jnp.where(valid, sg_abs, 1.0)

    hit = (u < sg) & valid
    sg_ref[:] = jnp.min(jnp.where(hit, lane, C - 1), axis=1, keepdims=True)

    def scatter(t, carry):
        r = idx_ref[t]
        iw_ref[pl.ds(r, 1), :] = iwg_ref[pl.ds(t, 1), :]
        samp_ref[pl.ds(r, 1), :] = sg_ref[pl.ds(t, 1), :]
        return carry

    jax.lax.fori_loop(0, KCAP, scatter, 0)


@jax.jit
def kernel(encoder_logits, u_noise):
    xp = encoder_logits.reshape(RP, W)
    up = u_noise.reshape(RP, W)
    iw1, samp1, flags = pl.pallas_call(
        _fast_kernel,
        out_shape=(
            jax.ShapeDtypeStruct((RP, W), jnp.float32),
            jax.ShapeDtypeStruct((RP, 2), jnp.int32),
            jax.ShapeDtypeStruct((RP, 2), jnp.int32),
        ),
    )(xp, up)
    idx = jnp.nonzero(flags.reshape(R), size=KCAP, fill_value=0)[0]
    idx = idx.astype(jnp.int32)
    iw, samp = pl.pallas_call(
        _fix_kernel,
        in_specs=[
            pl.BlockSpec(memory_space=pltpu.SMEM),
            pl.BlockSpec(memory_space=pltpu.VMEM),
            pl.BlockSpec(memory_space=pltpu.VMEM),
            pl.BlockSpec(memory_space=pltpu.VMEM),
            pl.BlockSpec(memory_space=pltpu.VMEM),
        ],
        out_specs=(
            pl.BlockSpec(memory_space=pltpu.VMEM),
            pl.BlockSpec(memory_space=pltpu.VMEM),
        ),
        out_shape=(
            jax.ShapeDtypeStruct((R, C), jnp.float32),
            jax.ShapeDtypeStruct((R, 1), jnp.int32),
        ),
        scratch_shapes=[
            pltpu.VMEM((KCAP, C), jnp.float32),
            pltpu.VMEM((KCAP, C), jnp.float32),
            pltpu.VMEM((KCAP, C), jnp.float32),
            pltpu.VMEM((KCAP, 1), jnp.int32),
        ],
    )(idx, encoder_logits.reshape(R, C), u_noise.reshape(R, C),
      iw1.reshape(R, C), samp1.reshape(R, 1))
    return iw.reshape(B, V, C), samp.reshape(B, V)


# speedup vs baseline: 4.3481x; 2.2941x over previous
import jax
import jax.numpy as jnp
from jax.experimental import pallas as pl

B, V, C = 64, 32, 64
R = B * V
RP = R // 2
W = 2 * C
KCAP = 128
_SHIFT = 16.0
_NEG = -jnp.inf


def _seg_shl1(x, pad):
    lane = jax.lax.broadcasted_iota(jnp.int32, x.shape, 1)
    y = jnp.concatenate(
        [x[:, 1:], jnp.full((x.shape[0], 1), pad, x.dtype)], axis=1)
    return jnp.where((lane & (C - 1)) == C - 1, pad, y)


def _seg_shr1(x, pad):
    lane = jax.lax.broadcasted_iota(jnp.int32, x.shape, 1)
    y = jnp.concatenate(
        [jnp.full((x.shape[0], 1), pad, x.dtype), x[:, :-1]], axis=1)
    return jnp.where((lane & (C - 1)) == 0, pad, y)


def _shr(x, k, pad):
    return jnp.concatenate(
        [jnp.full((x.shape[0], k), pad, x.dtype), x[:, : x.shape[1] - k]],
        axis=1)


def _mm(a, b, precision):
    return jax.lax.dot_general(
        a, b, (((1,), (0,)), ((), ())),
        preferred_element_type=jnp.float32, precision=precision)


def _mmt(a, b, precision):
    return jax.lax.dot_general(
        a, b, (((0,), (0,)), ((), ())),
        preferred_element_type=jnp.float32, precision=precision)


def _kernel(x_ref, u_ref, iw_ref, s_ref):
    x = x_ref[:]
    u = u_ref[:]
    lane = jax.lax.broadcasted_iota(jnp.int32, (RP, W), 1)
    valid = (lane & (C - 1)) < C - 1

    jj = jax.lax.broadcasted_iota(jnp.int32, (W, W), 0)
    ii = jax.lax.broadcasted_iota(jnp.int32, (W, W), 1)
    samehalf = (jj // C) == (ii // C)
    upper = ((jj > ii) & samehalf).astype(jnp.float32)
    lower_f = ((jj <= ii) & samehalf).astype(jnp.float32)
    lower_b = ((jj <= ii) & samehalf).astype(jnp.bfloat16)

    e = jnp.exp(x - _SHIFT)
    s = _mm(e, upper, jax.lax.Precision.HIGHEST)
    xs = _seg_shl1(x, 0.0)
    denom = xs + jnp.log(s * jnp.exp(_SHIFT - xs))
    sb = jnp.where(valid, x - denom, 0.0)

    sg = jax.nn.sigmoid(sb)
    sg_abs = jnp.maximum(sg, 1.0 - sg)
    cond = sb >= -1e-5
    ratio = (1.0 - sg) ** 2 / jnp.where(cond, 1.0, 1.0 - 2.0 * sg)
    lg = jnp.where(cond, -1e4, jnp.log(ratio))
    p = jnp.exp(_mm(lg, lower_f, jax.lax.Precision.HIGHEST))
    iw_fast = _seg_shr1(p, 1.0) * jnp.where(valid, sg_abs, 1.0)

    h = (u < sg).astype(jnp.bfloat16)
    q = _mm(h, lower_b, jax.lax.Precision.DEFAULT)
    ind = (q == 0.0).astype(jnp.bfloat16)
    oj = jax.lax.broadcasted_iota(jnp.int32, (W, 2), 0)
    oc = jax.lax.broadcasted_iota(jnp.int32, (W, 2), 1)
    seg_of = ((oj // C) == oc)
    count = (seg_of & ((oj & (C - 1)) < C - 1)).astype(jnp.bfloat16)
    s_fast = _mm(ind, count, jax.lax.Precision.DEFAULT).astype(jnp.int32)

    danger = valid & (((sb > -2e-2) & (sb < 9e-5))
                      | (jnp.abs(u - sg) < 1e-5))
    dcnt = _mm(danger.astype(jnp.bfloat16), seg_of.astype(jnp.bfloat16),
               jax.lax.Precision.DEFAULT)
    fseg = dcnt > 0.0

    pp = jax.lax.broadcasted_iota(jnp.int32, (RP, RP), 0)
    qq = jax.lax.broadcasted_iota(jnp.int32, (RP, RP), 1)
    ltri = (qq < pp).astype(jnp.bfloat16)
    fb = fseg.astype(jnp.bfloat16)
    pre = _mm(ltri, fb, jax.lax.Precision.DEFAULT)
    tot0 = jnp.sum(fseg[:, 0:1].astype(jnp.float32))
    lane2 = jax.lax.broadcasted_iota(jnp.int32, (RP, 2), 1)
    rank = pre + jnp.where(lane2 == 1, tot0, 0.0)

    rank_i = rank.astype(jnp.int32)
    tl = jax.lax.broadcasted_iota(jnp.int32, (RP, KCAP), 1)
    in_cap = rank_i < KCAP
    fseg_f = fseg.astype(jnp.float32)
    selt_e = (rank_i[:, 0:1] == tl).astype(jnp.float32) * fseg_f[:, 0:1]
    selt_o = (rank_i[:, 1:2] == tl).astype(jnp.float32) * fseg_f[:, 1:2]

    ge = _mmt(selt_e, x, jax.lax.Precision.HIGHEST)
    go = _mmt(selt_o, x, jax.lax.Precision.HIGHEST)
    xg = ge[:, :C] + go[:, C:]
    ue = _mmt(selt_e, u, jax.lax.Precision.HIGHEST)
    uo = _mmt(selt_o, u, jax.lax.Precision.HIGHEST)
    ug = ue[:, :C] + uo[:, C:]

    glane = jax.lax.broadcasted_iota(jnp.int32, (KCAP, C), 1)
    gvalid = glane < C - 1

    tiled = jnp.broadcast_to(xg[:, None, :], (KCAP, C, C))
    gii = jax.lax.broadcasted_iota(jnp.int32, (KCAP, C, C), 1)
    gjj = jax.lax.broadcasted_iota(jnp.int32, (KCAP, C, C), 2)
    dl = jnp.where(gjj <= gii, _NEG, tiled)
    amax = jnp.max(dl, axis=-1, keepdims=True)
    ssum = jnp.sum(jnp.exp(dl - amax), axis=-1)
    gdenom = jnp.log(ssum) + amax[:, :, 0]
    gsb = jnp.where(gvalid, xg - gdenom, 0.0)

    gsg = jax.nn.sigmoid(gsb)
    gsg_abs = jax.nn.sigmoid(jnp.abs(gsb))
    gcond = gsb >= -1e-5
    safe_den = jnp.where(gcond, 1.0, 1.0 - 2.0 * gsg)
    bzm = jnp.where(gcond, 0.0, (1.0 - gsg) ** 2 / safe_den)
    bzm = jnp.where(gvalid, bzm, 0.0)
    for k in (1, 2, 4, 8, 16, 32):
        bzm = bzm * _shr(bzm, k, 1.0)
    iw_fix = _shr(bzm, 1, 1.0) * jnp.where(gvalid, gsg_abs, 1.0)

    ghit = (ug < gsg) & gvalid
    s_fix = jnp.min(jnp.where(ghit, glane, C - 1), axis=1, keepdims=True)
    s_fix_f = s_fix.astype(jnp.float32)

    zpad = jnp.zeros((KCAP, C), jnp.float32)
    scat = (_mm(selt_e, jnp.concatenate([iw_fix, zpad], axis=1),
                jax.lax.Precision.HIGHEST)
            + _mm(selt_o, jnp.concatenate([zpad, iw_fix], axis=1),
                  jax.lax.Precision.HIGHEST))
    s_scat = jnp.concatenate(
        [_mm(selt_e, s_fix_f, jax.lax.Precision.HIGHEST),
         _mm(selt_o, s_fix_f, jax.lax.Precision.HIGHEST)], axis=1)

    fixed = fseg & in_cap
    fixed_f = fixed.astype(jnp.float32)
    is_lo = (lane < C).astype(jnp.float32)
    segmask_f = fixed_f[:, 0:1] * is_lo + fixed_f[:, 1:2] * (1.0 - is_lo)
    iw_ref[:] = jnp.where(segmask_f > 0.5, scat, iw_fast)
    s_ref[:] = jnp.where(fixed, s_scat.astype(jnp.int32), s_fast)


@jax.jit
def kernel(encoder_logits, u_noise):
    xp = encoder_logits.reshape(RP, W)
    up = u_noise.reshape(RP, W)
    iw, samp = pl.pallas_call(
        _kernel,
        out_shape=(
            jax.ShapeDtypeStruct((RP, W), jnp.float32),
            jax.ShapeDtypeStruct((RP, 2), jnp.int32),
        ),
    )(xp, up)
    return iw.reshape(B, V, C), samp.reshape(B, V)


# speedup vs baseline: 4.3991x; 1.0117x over previous
import jax
import jax.numpy as jnp
from jax.experimental import pallas as pl

B, V, C = 64, 32, 64
R = B * V
RP = R // 2
W = 2 * C
KCAP = 128
_SHIFT = 16.0
_NEG = -jnp.inf


def _seg_shl1(x, pad):
    lane = jax.lax.broadcasted_iota(jnp.int32, x.shape, 1)
    y = jnp.concatenate(
        [x[:, 1:], jnp.full((x.shape[0], 1), pad, x.dtype)], axis=1)
    return jnp.where((lane & (C - 1)) == C - 1, pad, y)


def _seg_shr1(x, pad):
    lane = jax.lax.broadcasted_iota(jnp.int32, x.shape, 1)
    y = jnp.concatenate(
        [jnp.full((x.shape[0], 1), pad, x.dtype), x[:, :-1]], axis=1)
    return jnp.where((lane & (C - 1)) == 0, pad, y)


def _shr(x, k, pad):
    return jnp.concatenate(
        [jnp.full((x.shape[0], k), pad, x.dtype), x[:, : x.shape[1] - k]],
        axis=1)


def _mm(a, b, precision):
    return jax.lax.dot_general(
        a, b, (((1,), (0,)), ((), ())),
        preferred_element_type=jnp.float32, precision=precision)


def _mmt(a, b, precision):
    return jax.lax.dot_general(
        a, b, (((0,), (0,)), ((), ())),
        preferred_element_type=jnp.float32, precision=precision)


def _kernel(x_ref, u_ref, iw_ref, s_ref):
    x = x_ref[:]
    u = u_ref[:]
    lane = jax.lax.broadcasted_iota(jnp.int32, (RP, W), 1)
    valid = (lane & (C - 1)) < C - 1

    jj = jax.lax.broadcasted_iota(jnp.int32, (W, W), 0)
    ii = jax.lax.broadcasted_iota(jnp.int32, (W, W), 1)
    samehalf = (jj // C) == (ii // C)
    upper = ((jj > ii) & samehalf).astype(jnp.float32)
    lower_f = ((jj <= ii) & samehalf).astype(jnp.float32)
    lower_b = ((jj <= ii) & samehalf).astype(jnp.bfloat16)

    e = jnp.exp(x - _SHIFT)
    s = _mm(e, upper, jax.lax.Precision.HIGHEST)
    xs = _seg_shl1(x, 0.0)
    denom = xs + jnp.log(s * jnp.exp(_SHIFT - xs))
    sb = jnp.where(valid, x - denom, 0.0)

    sg = jax.nn.sigmoid(sb)
    sg_abs = jnp.maximum(sg, 1.0 - sg)
    cond = sb >= -1e-5
    ratio = (1.0 - sg) ** 2 / jnp.where(cond, 1.0, 1.0 - 2.0 * sg)
    lg = jnp.where(cond, -1e4, jnp.log(ratio))
    p = jnp.exp(_mm(lg, lower_f, jax.lax.Precision.HIGHEST))
    iw_fast = _seg_shr1(p, 1.0) * jnp.where(valid, sg_abs, 1.0)

    h = (u < sg).astype(jnp.bfloat16)
    q = _mm(h, lower_b, jax.lax.Precision.DEFAULT)
    ind = (q == 0.0).astype(jnp.bfloat16)
    oj = jax.lax.broadcasted_iota(jnp.int32, (W, 2), 0)
    oc = jax.lax.broadcasted_iota(jnp.int32, (W, 2), 1)
    seg_of = ((oj // C) == oc)
    count = (seg_of & ((oj & (C - 1)) < C - 1)).astype(jnp.bfloat16)
    s_fast = _mm(ind, count, jax.lax.Precision.DEFAULT).astype(jnp.int32)

    danger = valid & (((sb > -2e-2) & (sb < 9e-5))
                      | (jnp.abs(u - sg) < 1e-5))
    dcnt = _mm(danger.astype(jnp.bfloat16), seg_of.astype(jnp.bfloat16),
               jax.lax.Precision.DEFAULT)
    fseg = dcnt > 0.0

    pp = jax.lax.broadcasted_iota(jnp.int32, (RP, RP), 0)
    qq = jax.lax.broadcasted_iota(jnp.int32, (RP, RP), 1)
    ltri = (qq < pp).astype(jnp.bfloat16)
    fb = fseg.astype(jnp.bfloat16)
    pre = _mm(ltri, fb, jax.lax.Precision.DEFAULT)
    tot0 = jnp.sum(fseg[:, 0:1].astype(jnp.float32))
    lane2 = jax.lax.broadcasted_iota(jnp.int32, (RP, 2), 1)
    rank = pre + jnp.where(lane2 == 1, tot0, 0.0)

    rank_i = rank.astype(jnp.int32)
    tl = jax.lax.broadcasted_iota(jnp.int32, (RP, KCAP), 1)
    in_cap = rank_i < KCAP
    fseg_f = fseg.astype(jnp.float32)
    selt_e = (rank_i[:, 0:1] == tl).astype(jnp.float32) * fseg_f[:, 0:1]
    selt_o = (rank_i[:, 1:2] == tl).astype(jnp.float32) * fseg_f[:, 1:2]

    selt_eo = jnp.concatenate([selt_e, selt_o], axis=1)
    xu = jnp.concatenate([x, u], axis=1)
    geo = _mmt(selt_eo, xu, jax.lax.Precision.HIGHEST)
    ge = geo[:KCAP]
    go = geo[KCAP:]
    xg = ge[:, :C] + go[:, C:W]
    ug = ge[:, W:W + C] + go[:, W + C:]

    glane = jax.lax.broadcasted_iota(jnp.int32, (KCAP, C), 1)
    gvalid = glane < C - 1

    tiled = jnp.broadcast_to(xg[:, None, :], (KCAP, C, C))
    gii = jax.lax.broadcasted_iota(jnp.int32, (KCAP, C, C), 1)
    gjj = jax.lax.broadcasted_iota(jnp.int32, (KCAP, C, C), 2)
    dl = jnp.where(gjj <= gii, _NEG, tiled)
    amax = jnp.max(dl, axis=-1, keepdims=True)
    ssum = jnp.sum(jnp.exp(dl - amax), axis=-1)
    gdenom = jnp.log(ssum) + amax[:, :, 0]
    gsb = jnp.where(gvalid, xg - gdenom, 0.0)

    gsg = jax.nn.sigmoid(gsb)
    gsg_abs = jax.nn.sigmoid(jnp.abs(gsb))
    gcond = gsb >= -1e-5
    safe_den = jnp.where(gcond, 1.0, 1.0 - 2.0 * gsg)
    glg = jnp.where(gcond | jnp.logical_not(gvalid), -1e4,
                    jnp.log((1.0 - gsg) ** 2 / safe_den))
    cii = jax.lax.broadcasted_iota(jnp.int32, (C, C), 1)
    cjj = jax.lax.broadcasted_iota(jnp.int32, (C, C), 0)
    strict_lower = (cjj < cii).astype(jnp.float32)
    iw_fix = (jnp.exp(_mm(glg, strict_lower, jax.lax.Precision.HIGHEST))
              * jnp.where(gvalid, gsg_abs, 1.0))

    ghit = (ug < gsg) & gvalid
    s_fix = jnp.min(jnp.where(ghit, glane, C - 1), axis=1, keepdims=True)
    s_fix_f = s_fix.astype(jnp.float32)

    zpad = jnp.zeros((KCAP, C), jnp.float32)
    zs = jnp.zeros((KCAP, 1), jnp.float32)
    stacked = jnp.concatenate(
        [jnp.concatenate([iw_fix, zpad], axis=1),
         jnp.concatenate([zpad, iw_fix], axis=1)], axis=0)
    scat = _mm(selt_eo, stacked, jax.lax.Precision.HIGHEST)
    stacked_s = jnp.concatenate(
        [jnp.concatenate([s_fix_f, zs], axis=1),
         jnp.concatenate([zs, s_fix_f], axis=1)], axis=0)
    s_scat = _mm(selt_eo, stacked_s, jax.lax.Precision.HIGHEST)

    fixed = fseg & in_cap
    fixed_f = fixed.astype(jnp.float32)
    is_lo = (lane < C).astype(jnp.float32)
    segmask_f = fixed_f[:, 0:1] * is_lo + fixed_f[:, 1:2] * (1.0 - is_lo)
    iw_ref[:] = jnp.where(segmask_f > 0.5, scat, iw_fast)
    s_ref[:] = jnp.where(fixed, s_scat.astype(jnp.int32), s_fast)


@jax.jit
def kernel(encoder_logits, u_noise):
    xp = encoder_logits.reshape(RP, W)
    up = u_noise.reshape(RP, W)
    iw, samp = pl.pallas_call(
        _kernel,
        out_shape=(
            jax.ShapeDtypeStruct((RP, W), jnp.float32),
            jax.ShapeDtypeStruct((RP, 2), jnp.int32),
        ),
    )(xp, up)
    return iw.reshape(B, V, C), samp.reshape(B, V)


# speedup vs baseline: 4.6023x; 1.0462x over previous
import jax
import jax.numpy as jnp
from jax.experimental import pallas as pl

B, V, C = 64, 32, 64
R = B * V
RH = R // 2
KCAP = 128
_SHIFT = 16.0
_NEG = -jnp.inf


def _mm(a, b, precision):
    return jax.lax.dot_general(
        a, b, (((1,), (0,)), ((), ())),
        preferred_element_type=jnp.float32, precision=precision)


def _mmt(a, b, precision):
    return jax.lax.dot_general(
        a, b, (((0,), (0,)), ((), ())),
        preferred_element_type=jnp.float32, precision=precision)


def _kernel(x_ref, u_ref, iw_ref, s_ref):
    x = x_ref[:]
    u = u_ref[:]
    lane = jax.lax.broadcasted_iota(jnp.int32, (R, C), 1)
    valid = lane < C - 1

    cjj = jax.lax.broadcasted_iota(jnp.int32, (C, C), 0)
    cii = jax.lax.broadcasted_iota(jnp.int32, (C, C), 1)
    upper = (cjj > cii).astype(jnp.float32)
    strict_lower = (cjj < cii).astype(jnp.float32)
    lower_b = (cjj <= cii).astype(jnp.bfloat16)

    e = jnp.exp(x - _SHIFT)
    s = _mm(e, upper, jax.lax.Precision.HIGHEST)
    xs = jnp.concatenate([x[:, 1:], jnp.zeros((R, 1), jnp.float32)], axis=1)
    denom = xs + jnp.log(s * jnp.exp(_SHIFT - xs))
    sb = jnp.where(valid, x - denom, 0.0)

    sg = jax.nn.sigmoid(sb)
    sg_abs = jnp.maximum(sg, 1.0 - sg)
    cond = sb >= -1e-5
    ratio = (1.0 - sg) ** 2 / jnp.where(cond, 1.0, 1.0 - 2.0 * sg)
    lg = jnp.where(cond, -1e4, jnp.log(ratio))
    iw_fast = (jnp.exp(_mm(lg, strict_lower, jax.lax.Precision.HIGHEST))
               * jnp.where(valid, sg_abs, 1.0))

    h = (u < sg).astype(jnp.bfloat16)
    q = _mm(h, lower_b, jax.lax.Precision.DEFAULT)
    ind = (q == 0.0).astype(jnp.bfloat16)
    cnt = (jax.lax.broadcasted_iota(jnp.int32, (C, 1), 0)
           < C - 1).astype(jnp.bfloat16)
    s_fast = _mm(ind, cnt, jax.lax.Precision.DEFAULT).astype(jnp.int32)

    danger = valid & (((sb > -2e-2) & (sb < 9e-5))
                      | (jnp.abs(u - sg) < 1e-5))
    f2 = (jnp.sum(danger.astype(jnp.float32).reshape(RH, 2, C), axis=2)
          > 0.0)
    f2f = f2.astype(jnp.float32)

    pp = jax.lax.broadcasted_iota(jnp.int32, (RH, RH), 0)
    qq = jax.lax.broadcasted_iota(jnp.int32, (RH, RH), 1)
    ltri = (qq < pp).astype(jnp.bfloat16)
    pre = _mm(ltri, f2.astype(jnp.bfloat16), jax.lax.Precision.DEFAULT)
    tot0 = jnp.sum(f2f[:, 0:1])
    lane2 = jax.lax.broadcasted_iota(jnp.int32, (RH, 2), 1)
    rank = (pre + jnp.where(lane2 == 1, tot0, 0.0)).astype(jnp.int32)

    tlh = jax.lax.broadcasted_iota(jnp.int32, (RH, KCAP), 1)
    se = (rank[:, 0:1] == tlh).astype(jnp.float32) * f2f[:, 0:1]
    so = (rank[:, 1:2] == tlh).astype(jnp.float32) * f2f[:, 1:2]
    selt = jnp.concatenate(
        [se[:, None, :], so[:, None, :]], axis=1).reshape(R, KCAP)

    xu = jnp.concatenate([x, u], axis=1)
    g = _mmt(selt, xu, jax.lax.Precision.HIGHEST)
    xg = g[:, :C]
    ug = g[:, C:]

    glane = jax.lax.broadcasted_iota(jnp.int32, (KCAP, C), 1)
    gvalid = glane < C - 1

    tiled = jnp.broadcast_to(xg[:, None, :], (KCAP, C, C))
    gii = jax.lax.broadcasted_iota(jnp.int32, (KCAP, C, C), 1)
    gjj = jax.lax.broadcasted_iota(jnp.int32, (KCAP, C, C), 2)
    dl = jnp.where(gjj <= gii, _NEG, tiled)
    amax = jnp.max(dl, axis=-1, keepdims=True)
    ssum = jnp.sum(jnp.exp(dl - amax), axis=-1)
    gdenom = jnp.log(ssum) + amax[:, :, 0]
    gsb = jnp.where(gvalid, xg - gdenom, 0.0)

    gsg = jax.nn.sigmoid(gsb)
    gsg_abs = jax.nn.sigmoid(jnp.abs(gsb))
    gcond = gsb >= -1e-5
    safe_den = jnp.where(gcond, 1.0, 1.0 - 2.0 * gsg)
    glg = jnp.where(gcond | jnp.logical_not(gvalid), -1e4,
                    jnp.log((1.0 - gsg) ** 2 / safe_den))
    iw_fix = (jnp.exp(_mm(glg, strict_lower, jax.lax.Precision.HIGHEST))
              * jnp.where(gvalid, gsg_abs, 1.0))

    ghit = (ug < gsg) & gvalid
    s_fix = jnp.min(jnp.where(ghit, glane, C - 1), axis=1, keepdims=True)
    s_fix_f = s_fix.astype(jnp.float32)

    scat = _mm(selt, iw_fix, jax.lax.Precision.HIGHEST)
    s_scat = _mm(selt, s_fix_f, jax.lax.Precision.HIGHEST)

    fixed_f = _mm(selt, jnp.ones((KCAP, 1), jnp.float32),
                  jax.lax.Precision.DEFAULT)
    mask_full = fixed_f * jnp.ones((R, C), jnp.float32)
    iw_ref[:] = jnp.where(mask_full > 0.5, scat, iw_fast)
    s_ref[:] = jnp.where(fixed_f > 0.5, s_scat.astype(jnp.int32), s_fast)


@jax.jit
def kernel(encoder_logits, u_noise):
    xr = encoder_logits.reshape(R, C)
    ur = u_noise.reshape(R, C)
    iw, samp = pl.pallas_call(
        _kernel,
        out_shape=(
            jax.ShapeDtypeStruct((R, C), jnp.float32),
            jax.ShapeDtypeStruct((R, 1), jnp.int32),
        ),
    )(xr, ur)
    return iw.reshape(B, V, C), samp.reshape(B, V)
